# Initial kernel scaffold; baseline (speedup 1.0000x reference)
import jax
import jax.numpy as jnp
from jax.experimental import pallas as pl


def kernel(points_heatmap):
    raise NotImplementedError("write your pallas kernel here")


# speedup vs baseline: 4.6410x; 4.6410x over previous
import jax
import jax.numpy as jnp
from jax.experimental import pallas as pl
from jax.experimental.pallas import tpu as pltpu

_K = 100
_CONF = 0.1
_NC = 80
_H = 128
_W = 128
_R = _NC * _H
_CAP = 128
_TSLOTS = 4
_BISECT = 27


def _fiota(shape, dim):
    return jax.lax.broadcasted_iota(jnp.int32, shape, dim).astype(jnp.float32)


def _shift_down(y, sh):
    return jnp.concatenate([jnp.zeros((sh, y.shape[1]), y.dtype), y[:-sh, :]],
                           axis=0)


def _body(x_ref, o_ref):
    f32 = jnp.float32
    x = x_ref[0]
    cls3 = x[:_NC]
    s3 = jax.nn.sigmoid(cls3)

    zc = jnp.zeros((_NC, _H, 1), f32)
    hm = jnp.maximum(s3, jnp.concatenate([zc, s3[:, :, :-1]], axis=2))
    hm = jnp.maximum(hm, jnp.concatenate([s3[:, :, 1:], zc], axis=2))
    zr = jnp.zeros((_NC, 1, _W), f32)
    vm = jnp.maximum(hm, jnp.concatenate([zr, hm[:, :-1, :]], axis=1))
    vm = jnp.maximum(vm, jnp.concatenate([hm[:, 1:, :], zr], axis=1))

    S3 = jnp.where(s3 == vm, s3, 0.0)
    S2 = S3.reshape(_R, _W)

    def bis(_, c):
        lo, hi = c
        mid = 0.5 * (lo + hi)
        cnt = jnp.sum((S2 > mid).astype(f32))
        take = cnt >= float(_K)
        return (jnp.where(take, mid, lo), jnp.where(take, hi, mid))

    lo, hi = jax.lax.fori_loop(0, _BISECT, bis, (f32(_CONF), f32(1.0)))

    ind = (S2 > lo)
    indf = ind.astype(f32)

    lane2 = _fiota((_R, _W), 1)
    strl = (_fiota((_W, _W), 0)
            < _fiota((_W, _W), 1)).astype(f32)
    ones128 = jnp.ones((_W, _W), f32)
    excl = jnp.dot(indf, strl, preferred_element_type=f32)
    rowtotB = jnp.dot(indf, ones128, preferred_element_type=f32)

    rbB = rowtotB
    sh = 1
    while sh < _R:
        rbB = rbB + _shift_down(rbB, sh)
        sh *= 2
    rbB = rbB - rowtotB
    m = jnp.sum(rowtotB) * (1.0 / _W)

    rowio = _fiota((_R, _W), 0)

    def ext(t, acc):
        acc_val, acc_row, acc_lane = acc
        tf = t.astype(f32)
        sel = jnp.logical_and(excl == tf, ind)
        valB = jnp.dot(jnp.where(sel, S2, 0.0), ones128,
                       preferred_element_type=f32,
                       precision=jax.lax.Precision.HIGHEST)
        laneB = jnp.dot(jnp.where(sel, lane2, 0.0), ones128,
                        preferred_element_type=f32)
        Ef = jnp.where(jnp.logical_and((rbB + tf) == lane2, rowtotB > tf),
                       1.0, 0.0)
        return (acc_val + jnp.sum(Ef * valB, axis=0, keepdims=True),
                acc_row + jnp.sum(Ef * rowio, axis=0, keepdims=True),
                acc_lane + jnp.sum(Ef * laneB, axis=0, keepdims=True))

    acc_val, acc_row, acc_lane = jax.lax.fori_loop(
        0, _TSLOTS, ext, (jnp.zeros((1, _CAP), f32),
                          jnp.zeros((1, _CAP), f32),
                          jnp.zeros((1, _CAP), f32)))

    kio = _fiota((1, _CAP), 1)
    val_s = jnp.where(kio < m, acc_val, -1.0)

    V = jnp.broadcast_to(val_s, (_CAP, _CAP))
    VT = V.T
    jio = _fiota((_CAP, _CAP), 1)
    iio = _fiota((_CAP, _CAP), 0)
    G = jnp.logical_or(V > VT,
                       jnp.logical_and(V == VT, jio < iio)).astype(f32)
    rank = jnp.sum(G, axis=1, keepdims=True)

    P = (rank == _fiota((_CAP, _CAP), 1)).astype(f32)

    def by_rank(fld):
        FT = jnp.broadcast_to(fld, (_CAP, _CAP)).T
        return jnp.sum(P * FT, axis=0, keepdims=True)

    valR = by_rank(val_s)
    rowR = by_rank(acc_row)
    xR = by_rank(acc_lane)
    cR = jnp.floor(rowR * (1.0 / _H))
    yR = rowR - cR * _H

    sub128 = _fiota((_H, _CAP), 0)
    Mx = (jnp.broadcast_to(xR, (_W, _CAP)) == sub128).astype(f32)
    My = (jnp.broadcast_to(yR, (_H, _CAP)) == sub128).astype(f32)

    def at_yx(img):
        T1 = jnp.dot(img, Mx, preferred_element_type=f32,
                     precision=jax.lax.Precision.HIGHEST)
        return jnp.sum(T1 * My, axis=0, keepdims=True)

    off0 = at_yx(x[_NC])
    off1 = at_yx(x[_NC + 1])
    sz0 = at_yx(x[_NC + 2])
    sz1 = at_yx(x[_NC + 3])

    rio = _fiota((1, _CAP), 1)
    msk = jnp.logical_and(valR > _CONF, rio < float(_K)).astype(f32)
    rows = [valR * msk, cR * msk, (xR + off0) * msk, (yR + off1) * msk,
            sz0 * msk, sz1 * msk]
    o_ref[0] = jnp.concatenate(rows, axis=0)


def kernel(points_heatmap):
    B = points_heatmap.shape[0]
    raw = pl.pallas_call(
        _body,
        grid=(B,),
        in_specs=[pl.BlockSpec((1, _NC + 4, _H, _W), lambda b: (b, 0, 0, 0))],
        out_specs=pl.BlockSpec((1, 6, _CAP), lambda b: (b, 0, 0)),
        out_shape=jax.ShapeDtypeStruct((B, 6, _CAP), jnp.float32),
        compiler_params=pltpu.CompilerParams(
            dimension_semantics=("arbitrary",),
            vmem_limit_bytes=120 * 1024 * 1024),
    )(points_heatmap)
    return jnp.transpose(raw, (0, 2, 1))[:, :_K, :]


# speedup vs baseline: 5.7067x; 1.2296x over previous
import jax
import jax.numpy as jnp
from jax.experimental import pallas as pl
from jax.experimental.pallas import tpu as pltpu

_K = 100
_CONF = 0.1
_NC = 80
_H = 128
_W = 128
_R = _NC * _H
_CAP = 128
_TSLOTS = 4
_BISECT = 27


def _fiota(shape, dim):
    return jax.lax.broadcasted_iota(jnp.int32, shape, dim).astype(jnp.float32)


def _shift_down(y, sh):
    return jnp.concatenate([jnp.zeros((sh, y.shape[1]), y.dtype), y[:-sh, :]],
                           axis=0)


def _body(x_ref, o_ref):
    f32 = jnp.float32
    x = x_ref[0]
    cls3 = x[:_NC]
    s3 = jax.nn.sigmoid(cls3)

    zc = jnp.zeros((_NC, _H, 1), f32)
    hm = jnp.maximum(s3, jnp.concatenate([zc, s3[:, :, :-1]], axis=2))
    hm = jnp.maximum(hm, jnp.concatenate([s3[:, :, 1:], zc], axis=2))
    zr = jnp.zeros((_NC, 1, _W), f32)
    vm = jnp.maximum(hm, jnp.concatenate([zr, hm[:, :-1, :]], axis=1))
    vm = jnp.maximum(vm, jnp.concatenate([hm[:, 1:, :], zr], axis=1))

    S3 = jnp.where(s3 == vm, s3, 0.0)
    S2 = S3.reshape(_R, _W)

    rm80 = jnp.max(S3, axis=2)

    def bisr(_, c):
        lo, hi = c
        mid = 0.5 * (lo + hi)
        cnt = jnp.sum((rm80 > mid).astype(f32))
        take = cnt >= float(_K)
        return (jnp.where(take, mid, lo), jnp.where(take, hi, mid))

    wlo, _ = jax.lax.fori_loop(0, _BISECT, bisr, (f32(_CONF), f32(1.0)))
    gmax = jnp.max(rm80)

    def cond(c):
        _, _, cl, it = c
        return jnp.logical_and(cl > float(_CAP - 4), it < 34.0)

    def body(c):
        lo, hi, cl, it = c
        mid = 0.5 * (lo + hi)
        cm = jnp.sum((S2 > mid).astype(f32))
        take = cm >= float(_K)
        return (jnp.where(take, mid, lo), jnp.where(take, hi, mid),
                jnp.where(take, cm, cl), it + 1.0)

    cl0 = jnp.sum((S2 > wlo).astype(f32))
    lo, _, _, _ = jax.lax.while_loop(cond, body, (wlo, gmax, cl0, f32(0.0)))

    ind = (S2 > lo)
    indf = ind.astype(f32)

    lane2 = _fiota((_R, _W), 1)
    strl = (_fiota((_W, _W), 0)
            < _fiota((_W, _W), 1)).astype(f32)
    ones128 = jnp.ones((_W, _W), f32)
    excl = jnp.dot(indf, strl, preferred_element_type=f32)
    rowtotB = jnp.dot(indf, ones128, preferred_element_type=f32)

    rbB = rowtotB
    sh = 1
    while sh < _R:
        rbB = rbB + _shift_down(rbB, sh)
        sh *= 2
    rbB = rbB - rowtotB
    m = jnp.sum(rowtotB) * (1.0 / _W)

    rowio = _fiota((_R, _W), 0)

    def ext(t, acc):
        acc_val, acc_row, acc_lane = acc
        tf = t.astype(f32)
        sel = jnp.logical_and(excl == tf, ind)
        valB = jnp.dot(jnp.where(sel, S2, 0.0), ones128,
                       preferred_element_type=f32,
                       precision=jax.lax.Precision.HIGHEST)
        laneB = jnp.dot(jnp.where(sel, lane2, 0.0), ones128,
                        preferred_element_type=f32)
        Ef = jnp.where(jnp.logical_and((rbB + tf) == lane2, rowtotB > tf),
                       1.0, 0.0)
        return (acc_val + jnp.sum(Ef * valB, axis=0, keepdims=True),
                acc_row + jnp.sum(Ef * rowio, axis=0, keepdims=True),
                acc_lane + jnp.sum(Ef * laneB, axis=0, keepdims=True))

    acc_val, acc_row, acc_lane = jax.lax.fori_loop(
        0, _TSLOTS, ext, (jnp.zeros((1, _CAP), f32),
                          jnp.zeros((1, _CAP), f32),
                          jnp.zeros((1, _CAP), f32)))

    kio = _fiota((1, _CAP), 1)
    val_s = jnp.where(kio < m, acc_val, -1.0)

    V = jnp.broadcast_to(val_s, (_CAP, _CAP))
    VT = V.T
    jio = _fiota((_CAP, _CAP), 1)
    iio = _fiota((_CAP, _CAP), 0)
    G = jnp.logical_or(V > VT,
                       jnp.logical_and(V == VT, jio < iio)).astype(f32)
    rank = jnp.sum(G, axis=1, keepdims=True)

    P = (rank == _fiota((_CAP, _CAP), 1)).astype(f32)

    def by_rank(fld):
        FT = jnp.broadcast_to(fld, (_CAP, _CAP)).T
        return jnp.sum(P * FT, axis=0, keepdims=True)

    valR = by_rank(val_s)
    rowR = by_rank(acc_row)
    xR = by_rank(acc_lane)
    cR = jnp.floor(rowR * (1.0 / _H))
    yR = rowR - cR * _H

    sub128 = _fiota((_H, _CAP), 0)
    Mx = (jnp.broadcast_to(xR, (_W, _CAP)) == sub128).astype(f32)
    My = (jnp.broadcast_to(yR, (_H, _CAP)) == sub128).astype(f32)

    def at_yx(img):
        T1 = jnp.dot(img, Mx, preferred_element_type=f32,
                     precision=jax.lax.Precision.HIGHEST)
        return jnp.sum(T1 * My, axis=0, keepdims=True)

    off0 = at_yx(x[_NC])
    off1 = at_yx(x[_NC + 1])
    sz0 = at_yx(x[_NC + 2])
    sz1 = at_yx(x[_NC + 3])

    rio = _fiota((1, _CAP), 1)
    msk = jnp.logical_and(valR > _CONF, rio < float(_K)).astype(f32)
    rows = [valR * msk, cR * msk, (xR + off0) * msk, (yR + off1) * msk,
            sz0 * msk, sz1 * msk]
    o_ref[0] = jnp.concatenate(rows, axis=0)


def kernel(points_heatmap):
    B = points_heatmap.shape[0]
    raw = pl.pallas_call(
        _body,
        grid=(B,),
        in_specs=[pl.BlockSpec((1, _NC + 4, _H, _W), lambda b: (b, 0, 0, 0))],
        out_specs=pl.BlockSpec((1, 6, _CAP), lambda b: (b, 0, 0)),
        out_shape=jax.ShapeDtypeStruct((B, 6, _CAP), jnp.float32),
        compiler_params=pltpu.CompilerParams(
            dimension_semantics=("arbitrary",),
            vmem_limit_bytes=120 * 1024 * 1024),
    )(points_heatmap)
    return jnp.transpose(raw, (0, 2, 1))[:, :_K, :]


# speedup vs baseline: 5.8574x; 1.0264x over previous
import jax
import jax.numpy as jnp
from jax.experimental import pallas as pl
from jax.experimental.pallas import tpu as pltpu

_K = 100
_CONF = 0.1
_NC = 80
_H = 128
_W = 128
_R = _NC * _H
_CAP = 128
_TSLOTS = 4
_BISECT = 27


def _fiota(shape, dim):
    return jax.lax.broadcasted_iota(jnp.int32, shape, dim).astype(jnp.float32)


def _shift_down(y, sh):
    return jnp.concatenate([jnp.zeros((sh, y.shape[1]), y.dtype), y[:-sh, :]],
                           axis=0)


def _body(x_ref, o_ref):
    f32 = jnp.float32
    x = x_ref[0]
    cls3 = x[:_NC]
    s3 = jax.nn.sigmoid(cls3)

    zc = jnp.zeros((_NC, _H, 1), f32)
    hm = jnp.maximum(s3, jnp.concatenate([zc, s3[:, :, :-1]], axis=2))
    hm = jnp.maximum(hm, jnp.concatenate([s3[:, :, 1:], zc], axis=2))
    zr = jnp.zeros((_NC, 1, _W), f32)
    vm = jnp.maximum(hm, jnp.concatenate([zr, hm[:, :-1, :]], axis=1))
    vm = jnp.maximum(vm, jnp.concatenate([hm[:, 1:, :], zr], axis=1))

    S3 = jnp.where(s3 == vm, s3, 0.0)
    S2 = S3.reshape(_R, _W)

    rm80 = jnp.max(S3, axis=2)

    def bisr(_, c):
        lo, hi = c
        mid = 0.5 * (lo + hi)
        cnt = jnp.sum((rm80 > mid).astype(f32))
        take = cnt >= float(_K)
        return (jnp.where(take, mid, lo), jnp.where(take, hi, mid))

    wlo, _ = jax.lax.fori_loop(0, _BISECT, bisr, (f32(_CONF), f32(1.0)))
    gmax = jnp.max(rm80)

    def cond(c):
        _, _, cl, it = c
        return jnp.logical_and(cl > float(_CAP - 4), it < 34.0)

    def body(c):
        lo, hi, cl, it = c
        mid = 0.5 * (lo + hi)
        cm = jnp.sum((S2 > mid).astype(f32))
        take = cm >= float(_K)
        return (jnp.where(take, mid, lo), jnp.where(take, hi, mid),
                jnp.where(take, cm, cl), it + 1.0)

    cl0 = jnp.sum((S2 > wlo).astype(f32))
    lo, _, _, _ = jax.lax.while_loop(cond, body, (wlo, gmax, cl0, f32(0.0)))

    ind = (S2 > lo)
    indf = ind.astype(f32)

    lane2 = _fiota((_R, _W), 1)
    strl = (_fiota((_W, _W), 0)
            < _fiota((_W, _W), 1)).astype(f32)
    ones128 = jnp.ones((_W, _W), f32)
    excl = jnp.dot(indf, strl, preferred_element_type=f32)
    rowtotB = jnp.dot(indf, ones128, preferred_element_type=f32)

    rowtot80 = jnp.sum(indf.reshape(_NC, _H, _W), axis=2)
    lexcl80 = jnp.dot(rowtot80, strl, preferred_element_type=f32)
    bs80 = jnp.sum(rowtot80, axis=1, keepdims=True)
    cb = bs80
    sh = 1
    while sh < _NC:
        cb = cb + _shift_down(cb, sh)
        sh *= 2
    rowbase80 = lexcl80 + (cb - bs80)
    rbB = jnp.broadcast_to(rowbase80[:, :, None],
                           (_NC, _H, _W)).reshape(_R, _W)
    m = jnp.sum(rowtotB) * (1.0 / _W)

    rowio = _fiota((_R, _W), 0)

    def ext(t, acc):
        acc_val, acc_row, acc_lane = acc
        tf = t.astype(f32)
        sel = jnp.logical_and(excl == tf, ind)
        valB = jnp.dot(jnp.where(sel, S2, 0.0), ones128,
                       preferred_element_type=f32,
                       precision=jax.lax.Precision.HIGHEST)
        laneB = jnp.dot(jnp.where(sel, lane2, 0.0), ones128,
                        preferred_element_type=f32)
        Ef = jnp.where(jnp.logical_and((rbB + tf) == lane2, rowtotB > tf),
                       1.0, 0.0)
        return (acc_val + jnp.sum(Ef * valB, axis=0, keepdims=True),
                acc_row + jnp.sum(Ef * rowio, axis=0, keepdims=True),
                acc_lane + jnp.sum(Ef * laneB, axis=0, keepdims=True))

    acc_val, acc_row, acc_lane = jax.lax.fori_loop(
        0, _TSLOTS, ext, (jnp.zeros((1, _CAP), f32),
                          jnp.zeros((1, _CAP), f32),
                          jnp.zeros((1, _CAP), f32)))

    kio = _fiota((1, _CAP), 1)
    val_s = jnp.where(kio < m, acc_val, -1.0)

    V = jnp.broadcast_to(val_s, (_CAP, _CAP))
    VT = V.T
    jio = _fiota((_CAP, _CAP), 1)
    iio = _fiota((_CAP, _CAP), 0)
    G = jnp.logical_or(V > VT,
                       jnp.logical_and(V == VT, jio < iio)).astype(f32)
    rank = jnp.sum(G, axis=1, keepdims=True)

    P = (rank == _fiota((_CAP, _CAP), 1)).astype(f32)

    def by_rank(fld):
        FT = jnp.broadcast_to(fld, (_CAP, _CAP)).T
        return jnp.sum(P * FT, axis=0, keepdims=True)

    valR = by_rank(val_s)
    rowR = by_rank(acc_row)
    xR = by_rank(acc_lane)
    cR = jnp.floor(rowR * (1.0 / _H))
    yR = rowR - cR * _H

    sub128 = _fiota((_H, _CAP), 0)
    Mx = (jnp.broadcast_to(xR, (_W, _CAP)) == sub128).astype(f32)
    My = (jnp.broadcast_to(yR, (_H, _CAP)) == sub128).astype(f32)

    def at_yx(img):
        T1 = jnp.dot(img, Mx, preferred_element_type=f32,
                     precision=jax.lax.Precision.HIGHEST)
        return jnp.sum(T1 * My, axis=0, keepdims=True)

    off0 = at_yx(x[_NC])
    off1 = at_yx(x[_NC + 1])
    sz0 = at_yx(x[_NC + 2])
    sz1 = at_yx(x[_NC + 3])

    rio = _fiota((1, _CAP), 1)
    msk = jnp.logical_and(valR > _CONF, rio < float(_K)).astype(f32)
    rows = [valR * msk, cR * msk, (xR + off0) * msk, (yR + off1) * msk,
            sz0 * msk, sz1 * msk]
    o_ref[0] = jnp.concatenate(rows, axis=0)


def kernel(points_heatmap):
    B = points_heatmap.shape[0]
    raw = pl.pallas_call(
        _body,
        grid=(B,),
        in_specs=[pl.BlockSpec((1, _NC + 4, _H, _W), lambda b: (b, 0, 0, 0))],
        out_specs=pl.BlockSpec((1, 6, _CAP), lambda b: (b, 0, 0)),
        out_shape=jax.ShapeDtypeStruct((B, 6, _CAP), jnp.float32),
        compiler_params=pltpu.CompilerParams(
            dimension_semantics=("arbitrary",),
            vmem_limit_bytes=120 * 1024 * 1024),
    )(points_heatmap)
    return jnp.transpose(raw, (0, 2, 1))[:, :_K, :]
